# Initial kernel scaffold; baseline (speedup 1.0000x reference)
import jax
import jax.numpy as jnp
from jax.experimental import pallas as pl


def kernel(pos_l_emb, neg_l_emb, c_emb, pos_edge_index, neg_edge_index, l_mlp_w0, l_mlp_b0, l_mlp_w1, l_mlp_b1, l_mlp_w2, l_mlp_b2, c_mlp_w0, c_mlp_b0, c_mlp_w1, c_mlp_b1, c_mlp_w2, c_mlp_b2, l_wih, l_bih, l_whh, l_bhh, c_wih, c_bih, c_whh, c_bhh):
    raise NotImplementedError("write your pallas kernel here")


# speedup vs baseline: 5.1439x; 5.1439x over previous
import functools

import jax
import jax.numpy as jnp
from jax import lax
from jax.experimental import pallas as pl
from jax.experimental.pallas import tpu as pltpu
from jax.experimental.pallas import tpu_sc as plsc

EMB = 64
NV = 10000
NCL = 10000
NE = 160000
NROUND = 4
NCORES = 2
NSUB = 16
NW = NCORES * NSUB
CHUNK = 128
NCHUNK_TILE = 40
NCB = 10
EPAD = NSUB * NCB * 8 * CHUNK
ACC_R = 10112
ZROWS = ACC_R // NSUB
BROW = 1000
HIGHEST = lax.Precision.HIGHEST


def _dot_t(a, w):
    return lax.dot_general(a, w, (((1,), (1,)), ((), ())),
                           precision=HIGHEST, preferred_element_type=jnp.float32)


def _mlp3(x, w0, b0, w1, b1, w2, b2):
    h = jnp.maximum(_dot_t(x, w0) + b0, 0.0)
    h = jnp.maximum(_dot_t(h, w1) + b1, 0.0)
    return _dot_t(h, w2) + b2


def _full_spec(shape):
    nd = len(shape)
    return pl.BlockSpec(shape, lambda i, _n=nd: (0,) * _n)


def _sc_pass(msg, gidx, sidx, zrows):
    tbl_rows = msg.shape[0]
    base = (tbl_rows // NSUB) // 8 * 8
    rem = tbl_rows - base * NSUB
    mesh = plsc.VectorSubcoreMesh(core_axis_name="c", subcore_axis_name="s")
    out_type = jax.ShapeDtypeStruct((NCORES, ACC_R, EMB), jnp.float32)
    scratch = [
        pltpu.VMEM_SHARED((tbl_rows, EMB), jnp.float32),
        pltpu.VMEM_SHARED((ACC_R, EMB), jnp.float32),
        pltpu.VMEM((8, CHUNK), jnp.int32),
        pltpu.VMEM((8, CHUNK), jnp.int32),
        pltpu.VMEM((CHUNK, EMB), jnp.float32),
        pltpu.SemaphoreType.DMA,
    ]

    @functools.partial(
        pl.kernel, out_type=out_type, mesh=mesh, scratch_types=scratch,
        compiler_params=pltpu.CompilerParams(use_tc_tiling_on_sc=False))
    def body(msg_hbm, g_hbm, s_hbm, z_hbm, out_hbm,
             tbl, acc, gv, sv, rowbuf, sem):
        cid = lax.axis_index("c")
        sid = lax.axis_index("s")

        t0 = sid * base
        pltpu.sync_copy(msg_hbm.at[pl.ds(t0, base)], tbl.at[pl.ds(t0, base)])
        if rem:
            @pl.when(sid == 0)
            def _():
                pltpu.sync_copy(msg_hbm.at[pl.ds(base * NSUB, rem)],
                                tbl.at[pl.ds(base * NSUB, rem)])
        z0 = sid * ZROWS
        pltpu.sync_copy(z_hbm.at[pl.ds(z0, ZROWS)], acc.at[pl.ds(z0, ZROWS)])
        plsc.subcore_barrier()

        @pl.loop(0, NCB)
        def _(hb):
            pltpu.sync_copy(g_hbm.at[cid, sid, hb], gv)
            pltpu.sync_copy(s_hbm.at[cid, sid, hb], sv)

            @pl.loop(0, 8)
            def _(j):
                pltpu.async_copy(tbl.at[gv.at[j]], rowbuf, sem).wait()
                pltpu.sync_copy(rowbuf, acc.at[sv.at[j]], add=True)

        plsc.subcore_barrier()
        pltpu.sync_copy(acc.at[pl.ds(z0, ZROWS)],
                        out_hbm.at[cid, pl.ds(z0, ZROWS)])

    return body(msg, gidx, sidx, zrows)


def _mlp_call(x, w0, b0, w1, b1, w2, b2):
    n = x.shape[0]
    nb = n // BROW
    ws = (w0, b0, w1, b1, w2, b2)

    def body(x_ref, w0r, b0r, w1r, b1r, w2r, b2r, o_ref):
        o_ref[...] = _mlp3(x_ref[...], w0r[...], b0r[...], w1r[...],
                           b1r[...], w2r[...], b2r[...])

    return pl.pallas_call(
        body,
        grid=(nb,),
        in_specs=[pl.BlockSpec((BROW, EMB), lambda i: (i, 0))]
        + [_full_spec(w.shape) for w in ws],
        out_specs=pl.BlockSpec((BROW, EMB), lambda i: (i, 0)),
        out_shape=jax.ShapeDtypeStruct((n, EMB), jnp.float32),
    )(x, *ws)


def _lstm_body(msg_in, h, hf_or_none, c_old, wih, whh, b, with_msg, mlp_ws):
    wa = wih[:, :EMB]
    gates = _dot_t(msg_in, wa) if hf_or_none is None else (
        _dot_t(msg_in, wa) + _dot_t(hf_or_none, wih[:, EMB:]))
    gates = gates + _dot_t(h, whh) + b
    ig = jax.nn.sigmoid(gates[:, :EMB])
    fg = jax.nn.sigmoid(gates[:, EMB:2 * EMB])
    gg = jnp.tanh(gates[:, 2 * EMB:3 * EMB])
    og = jax.nn.sigmoid(gates[:, 3 * EMB:])
    c_new = fg * c_old + ig * gg
    h_new = og * jnp.tanh(c_new)
    msg = _mlp3(h_new, *mlp_ws) if with_msg else None
    return h_new, c_new, msg


def _lstm_l_call(l_h, l_c, c2l, wih, whh, b, mlp_ws, with_msg):
    nb = NV // BROW

    def body(h_ref, c_ref, cl_ref, wih_r, whh_r, b_r, *rest):
        if with_msg:
            (w0, b0, w1, b1, w2, b2, h_out, c_out, m_out) = rest
            mws = (w0[...], b0[...], w1[...], b1[...], w2[...], b2[...])
        else:
            (h_out, c_out) = rest
            mws = None
        hp, hn = h_ref[0], h_ref[1]
        x1 = jnp.concatenate([cl_ref[0], cl_ref[1]], axis=0)
        flip = jnp.concatenate([hn, hp], axis=0)
        h = jnp.concatenate([hp, hn], axis=0)
        c = jnp.concatenate([c_ref[0], c_ref[1]], axis=0)
        h_new, c_new, msg = _lstm_body(x1, h, flip, c, wih_r[...], whh_r[...],
                                       b_r[...], with_msg, mws)
        h_out[0], h_out[1] = h_new[:BROW], h_new[BROW:]
        c_out[0], c_out[1] = c_new[:BROW], c_new[BROW:]
        if with_msg:
            m_out[0], m_out[1] = msg[:BROW], msg[BROW:]

    pair = pl.BlockSpec((2, BROW, EMB), lambda i: (0, i, 0))
    w_in = [wih, whh, b]
    if with_msg:
        w_in += list(mlp_ws)
    out_shape = [jax.ShapeDtypeStruct((2, NV, EMB), jnp.float32)] * (3 if with_msg else 2)
    out_specs = [pair] * len(out_shape)
    return pl.pallas_call(
        body,
        grid=(nb,),
        in_specs=[pair, pair, pair] + [_full_spec(w.shape) for w in w_in],
        out_specs=out_specs,
        out_shape=out_shape,
    )(l_h, l_c, c2l, *w_in)


def _lstm_c_call(c_h, c_c, l2c_p, wih, whh, b, mlp_ws, with_msg):
    n = NCL
    nb = n // BROW

    def body(h_ref, c_ref, p_ref, wih_r, whh_r, b_r, *rest):
        if with_msg:
            (w0, b0, w1, b1, w2, b2, h_out, c_out, m_out) = rest
            mws = (w0[...], b0[...], w1[...], b1[...], w2[...], b2[...])
        else:
            (h_out, c_out) = rest
            mws = None
        x1 = p_ref[0] + p_ref[1]
        h_new, c_new, msg = _lstm_body(x1, h_ref[...], None, c_ref[...],
                                       wih_r[...], whh_r[...], b_r[...],
                                       with_msg, mws)
        h_out[...] = h_new
        c_out[...] = c_new
        if with_msg:
            m_out[...] = msg

    row = pl.BlockSpec((BROW, EMB), lambda i: (i, 0))
    part = pl.BlockSpec((2, BROW, EMB), lambda i: (0, i, 0))
    w_in = [wih, whh, b]
    if with_msg:
        w_in += list(mlp_ws)
    out_shape = [jax.ShapeDtypeStruct((n, EMB), jnp.float32)] * (3 if with_msg else 2)
    out_specs = [row] * len(out_shape)
    return pl.pallas_call(
        body,
        grid=(nb,),
        in_specs=[row, row, part] + [_full_spec(w.shape) for w in w_in],
        out_specs=out_specs,
        out_shape=out_shape,
    )(c_h, c_c, l2c_p, *w_in)


def _prep_edges(pos_edge_index, neg_edge_index):
    pos_src = pos_edge_index[0].astype(jnp.int32)
    pos_dst = pos_edge_index[1].astype(jnp.int32)
    neg_src = neg_edge_index[0].astype(jnp.int32)
    neg_dst = neg_edge_index[1].astype(jnp.int32)
    pad = EPAD - NE
    iot = lax.iota(jnp.int32, pad)
    spad = NV + iot % (ACC_R - NV)

    shp = (NSUB, NCB, 8, CHUNK)

    def mk(g, s, tbl_rows):
        gpad = iot % tbl_rows
        g = jnp.concatenate([g, gpad]).reshape(shp)
        s = jnp.concatenate([s, spad]).reshape(shp)
        return g, s

    g0, s0 = mk(pos_src, pos_dst, 2 * NV)
    g1, s1 = mk(neg_src + NV, neg_dst, 2 * NV)
    g2, s2 = mk(pos_dst, pos_src, NCL)
    g3, s3 = mk(neg_dst, neg_src, NCL)
    return ((jnp.stack([g0, g1]), jnp.stack([s0, s1])),
            (jnp.stack([g2, g3]), jnp.stack([s2, s3])))


def kernel(pos_l_emb, neg_l_emb, c_emb, pos_edge_index, neg_edge_index,
           l_mlp_w0, l_mlp_b0, l_mlp_w1, l_mlp_b1, l_mlp_w2, l_mlp_b2,
           c_mlp_w0, c_mlp_b0, c_mlp_w1, c_mlp_b1, c_mlp_w2, c_mlp_b2,
           l_wih, l_bih, l_whh, l_bhh, c_wih, c_bih, c_whh, c_bhh):
    l_ws = (l_mlp_w0, l_mlp_b0.reshape(1, EMB), l_mlp_w1,
            l_mlp_b1.reshape(1, EMB), l_mlp_w2, l_mlp_b2.reshape(1, EMB))
    c_ws = (c_mlp_w0, c_mlp_b0.reshape(1, EMB), c_mlp_w1,
            c_mlp_b1.reshape(1, EMB), c_mlp_w2, c_mlp_b2.reshape(1, EMB))
    l_b = (l_bih + l_bhh).reshape(1, 4 * EMB)
    c_b = (c_bih + c_bhh).reshape(1, 4 * EMB)

    (GA, SA), (GB, SB) = _prep_edges(pos_edge_index, neg_edge_index)
    zrows = jnp.zeros((ACC_R, EMB), jnp.float32)

    l_h = jnp.stack([pos_l_emb, neg_l_emb])
    l_c = jnp.zeros_like(l_h)
    c_h = c_emb
    c_c = jnp.zeros_like(c_h)

    l_msg = _mlp_call(l_h.reshape(2 * NV, EMB), *l_ws)
    c_msg = _mlp_call(c_h, *c_ws)
    for r in range(NROUND):
        l2c_p = _sc_pass(l_msg, GA, SA, zrows)
        c2l = _sc_pass(c_msg, GB, SB, zrows)
        wm = r < NROUND - 1
        lres = _lstm_l_call(l_h, l_c, c2l, l_wih, l_whh, l_b, l_ws, wm)
        cres = _lstm_c_call(c_h, c_c, l2c_p, c_wih, c_whh, c_b, c_ws, wm)
        if wm:
            l_h, l_c, l_msg = lres
            c_h, c_c, c_msg = cres
            l_msg = l_msg.reshape(2 * NV, EMB)
        else:
            l_h, l_c = lres
            c_h, c_c = cres
    return jnp.concatenate([l_h.reshape(2 * NV, EMB), c_h], axis=0)
